# baseline (speedup 1.0000x reference)
import functools
import jax
import jax.numpy as jnp
from jax import lax
from jax.experimental import pallas as pl
from jax.experimental.pallas import tpu as pltpu
from jax.experimental.pallas import tpu_sc as plsc

_K = 8192
_D = 768
_B = 4096
_ETA = 0.05
_EB = 2048
_MB = 2048
_NE = _B // _EB
_NM = _K // _MB
_STEPS = _NE + _NM

_NW = 16
_CH = _K // _NW
_NV = _CH // 16
_BIG = 2 ** 30


def _stream_body(eng_ref, m_ref,
                 mnew_ref, pmax_ref, pidx_ref, emean_ref,
                 eacc_ref, en_ref):
    i = pl.program_id(0)

    @pl.when(i == 0)
    def _():
        eacc_ref[...] = jnp.sum(eng_ref[...], axis=0, keepdims=True)

    @pl.when(jnp.logical_and(i > 0, i < _NE))
    def _():
        eacc_ref[...] += jnp.sum(eng_ref[...], axis=0, keepdims=True)

    @pl.when(i == _NE)
    def _():
        e = eacc_ref[...] / _B
        en_ref[...] = e / (jnp.sqrt(jnp.sum(e * e)) + 1e-6)

    @pl.when(i >= _NE)
    def _():
        c = i - _NE
        chunk = m_ref[...]
        mnew_ref[...] = chunk

        dot = jnp.dot(chunk, en_ref[...].reshape(_D, 1),
                      preferred_element_type=jnp.float32)
        nsq = jnp.sum(chunk * chunk, axis=1, keepdims=True)
        simc = dot * lax.rsqrt(jnp.maximum(nsq, 1e-24))

        lmax = jnp.max(simc)
        ii = lax.broadcasted_iota(jnp.int32, (_MB, 1), 0)
        lidx = jnp.min(jnp.where(simc == lmax, ii, _K)) + c * _MB
        pmax_ref[c] = lmax
        pidx_ref[c] = lidx

    @pl.when(i == _STEPS - 1)
    def _():
        emean_ref[...] = eacc_ref[...] / _B


def _sc_tail_body(pmax_hbm, pidx_hbm, usage_hbm, age_hbm,
                    idx_hbm, unew_hbm, anew_hbm,
                    u_v, a_v, tmpf_v, tmpi_v):
    w = lax.axis_index("s")

    pltpu.sync_copy(usage_hbm, u_v)
    pltpu.sync_copy(age_hbm, a_v)
    pltpu.sync_copy(pmax_hbm, tmpf_v)
    pltpu.sync_copy(pidx_hbm, tmpi_v)

    lane = lax.broadcasted_iota(jnp.int32, (16,), 0)

    def body(j, carry):
        worst, widx = carry
        u16 = u_v[pl.ds(j * 16, 16)]
        a16 = a_v[pl.ds(j * 16, 16)]
        sc = u16 - 0.01 * a16
        gi = lane + j * 16
        mw = sc < worst
        return jnp.where(mw, sc, worst), jnp.where(mw, gi, widx)

    worst, widx = lax.fori_loop(
        0, _K // 16, body,
        (jnp.full((16,), jnp.inf, jnp.float32),
         jnp.full((16,), _BIG, jnp.int32)))

    fmin = worst[0]
    fb = widx[0]
    for l in range(1, 16):
        bet = jnp.logical_or(
            worst[l] < fmin,
            jnp.logical_and(worst[l] == fmin, widx[l] < fb))
        fmin = jnp.where(bet, worst[l], fmin)
        fb = jnp.where(bet, widx[l], fb)

    pm = tmpf_v[...]
    pi = tmpi_v[...]
    gmax = pm[0]
    gidx = pi[0]
    for c in range(1, 4):
        better = pm[c] > gmax
        gmax = jnp.where(better, pm[c], gmax)
        gidx = jnp.where(better, pi[c], gidx)

    idx = jnp.where(gmax < 0.3, fb, gidx)
    idxv = jnp.full((16,), idx, jnp.int32)
    tmpi_v[...] = idxv
    pltpu.sync_copy(tmpi_v, idx_hbm)

    base = w * _CH
    for j in range(_NV):
        gi = lane + base + j * 16
        hot = (gi == idxv)
        u16 = u_v[pl.ds(base + j * 16, 16)]
        a16 = a_v[pl.ds(base + j * 16, 16)]
        u_v[pl.ds(base + j * 16, 16)] = \
            (u16 + jnp.where(hot, 1.0, 0.0)) * 0.999
        a_v[pl.ds(base + j * 16, 16)] = jnp.where(hot, 0.0, a16 + 1.0)

    pltpu.sync_copy(u_v.at[pl.ds(base, _CH)], unew_hbm.at[pl.ds(base, _CH)])
    pltpu.sync_copy(a_v.at[pl.ds(base, _CH)], anew_hbm.at[pl.ds(base, _CH)])


def _make_sc_tail():
    mesh = plsc.VectorSubcoreMesh(core_axis_name="c", subcore_axis_name="s",
                                  num_cores=1)
    return pl.kernel(
        _sc_tail_body,
        mesh=mesh,
        out_type=[
            jax.ShapeDtypeStruct((16,), jnp.int32),
            jax.ShapeDtypeStruct((_K,), jnp.float32),
            jax.ShapeDtypeStruct((_K,), jnp.float32),
        ],
        scratch_types=[
            pltpu.VMEM((_K,), jnp.float32),
            pltpu.VMEM((_K,), jnp.float32),
            pltpu.VMEM((16,), jnp.float32),
            pltpu.VMEM((16,), jnp.int32),
        ],
    )


def _fixup_body(idx_sref, mblk_ref, e_ref, out_ref):
    r = lax.rem(idx_sref[0], 8)
    out_ref[...] = mblk_ref[...]
    out_ref[pl.ds(r, 1), :] = ((1.0 - _ETA) * mblk_ref[pl.ds(r, 1), :]
                               + _ETA * e_ref[...])


def kernel(M, usage, age, engram):
    m_copy, pmax, pidx, e_mean = pl.pallas_call(
        _stream_body,
        grid=(_STEPS,),
        in_specs=[
            pl.BlockSpec((_EB, _D), lambda i: (jnp.minimum(i, _NE - 1), 0)),
            pl.BlockSpec((_MB, _D), lambda i: (jnp.maximum(i - _NE, 0), 0)),
        ],
        out_specs=[
            pl.BlockSpec((_MB, _D), lambda i: (jnp.maximum(i - _NE, 0), 0)),
            pl.BlockSpec(memory_space=pltpu.SMEM),
            pl.BlockSpec(memory_space=pltpu.SMEM),
            pl.BlockSpec((1, _D), lambda i: (0, 0)),
        ],
        out_shape=[
            jax.ShapeDtypeStruct((_K, _D), jnp.float32),
            jax.ShapeDtypeStruct((16,), jnp.float32),
            jax.ShapeDtypeStruct((16,), jnp.int32),
            jax.ShapeDtypeStruct((1, _D), jnp.float32),
        ],
        scratch_shapes=[
            pltpu.VMEM((1, _D), jnp.float32),
            pltpu.VMEM((1, _D), jnp.float32),
        ],
    )(engram, M)

    idx, u_new, a_new = _make_sc_tail()(pmax, pidx, usage, age)

    m_new = pl.pallas_call(
        _fixup_body,
        grid_spec=pltpu.PrefetchScalarGridSpec(
            num_scalar_prefetch=1,
            grid=(1,),
            in_specs=[
                pl.BlockSpec((8, _D), lambda i, idx_s: (idx_s[0] // 8, 0)),
                pl.BlockSpec((1, _D), lambda i, idx_s: (0, 0)),
            ],
            out_specs=pl.BlockSpec((8, _D), lambda i, idx_s: (idx_s[0] // 8, 0)),
        ),
        out_shape=jax.ShapeDtypeStruct((_K, _D), jnp.float32),
        input_output_aliases={1: 0},
    )(idx, m_copy, e_mean)

    return m_new, u_new, a_new
